# baseline (speedup 1.0000x reference)
import functools

import jax
import jax.numpy as jnp
from jax import lax
from jax.experimental import pallas as pl
from jax.experimental.pallas import tpu as pltpu
from jax.experimental.pallas import tpu_sc as plsc

VOCAB = 1000000
EMBED_DIM = 64
BATCH = 4096
HIST = 200

NUM_CORES = 2
NUM_SUBCORES = 16
NUM_WORKERS = NUM_CORES * NUM_SUBCORES

B_TOTAL = BATCH * HIST
B_PER_W = B_TOTAL // NUM_WORKERS
T = 256
N_CHUNKS = B_PER_W // T
NBUF = 2
LANES = 16
NSEG = 2 * T // 128
TRASH = T


def _make_gather():
  mesh = plsc.VectorSubcoreMesh(
      core_axis_name="c", subcore_axis_name="s",
      num_cores=NUM_CORES, num_subcores=NUM_SUBCORES)

  @functools.partial(
      pl.kernel,
      mesh=mesh,
      out_type=jax.ShapeDtypeStruct((B_TOTAL // 2, 2 * EMBED_DIM),
                                    jnp.float32),
      scratch_types=[
          pltpu.VMEM((B_PER_W,), jnp.int32),
          pltpu.VMEM((NBUF * T,), jnp.int32),
          pltpu.VMEM((NBUF * 2 * T,), jnp.int32),
          pltpu.VMEM((NBUF, T, 2 * EMBED_DIM), jnp.float32),
          [pltpu.SemaphoreType.DMA] * NBUF,
          [pltpu.SemaphoreType.DMA] * NBUF,
      ],
  )
  def gather_kernel(idx_hbm, table_hbm, out_hbm,
                    idx_v, pairidx, scatidx, buf128,
                    gsems, ssems):
    wid = lax.axis_index("s") * NUM_CORES + lax.axis_index("c")
    tok_base = wid * B_PER_W
    out_base = wid * (B_PER_W // 2)
    pltpu.sync_copy(idx_hbm.at[pl.ds(tok_base, B_PER_W)], idx_v)

    @pl.loop(0, N_CHUNKS, step=NBUF)
    def _group(g0):
      for b in range(NBUF):
        g = g0 + b
        for k in range(T // LANES):
          t16 = idx_v[pl.ds(g * T + k * LANES, LANES)]
          r16 = lax.iota(jnp.int32, LANES) + (k * LANES)
          par16 = t16 & 1
          pairidx[pl.ds(b * T + k * LANES, LANES)] = (
              lax.shift_right_logical(t16, 1))
          seg = (k * LANES) // 128
          off = (k * LANES) % 128
          trash16 = jnp.full((LANES,), TRASH, jnp.int32)
          scatidx[pl.ds(b * 2 * T + (2 * seg) * 128 + off, LANES)] = (
              jnp.where(par16 == 0, r16, trash16))
          scatidx[pl.ds(b * 2 * T + (2 * seg + 1) * 128 + off, LANES)] = (
              jnp.where(par16 == 1, r16, trash16))
        pltpu.async_copy(
            table_hbm.at[pairidx.at[pl.ds(b * T, T)]], buf128.at[b],
            gsems[b])

      for b in range(NBUF):
        g = g0 + b
        pltpu.make_async_copy(
            table_hbm.at[pl.ds(0, T)], buf128.at[b], gsems[b]).wait()

        @pl.when(g0 > 0)
        def _():
          pltpu.make_async_copy(
              buf128.at[b, pl.ds(0, T // 2)],
              out_hbm.at[pl.ds(out_base, T // 2)], ssems[b]).wait()

        pltpu.async_copy(
            buf128.at[b, pl.ds(0, T // 2)],
            out_hbm.at[pl.ds(out_base + g * (T // 2), T // 2)], ssems[b])

    for b in range(NBUF):
      pltpu.make_async_copy(
          buf128.at[b, pl.ds(0, T // 2)],
          out_hbm.at[pl.ds(out_base, T // 2)], ssems[b]).wait()

  return gather_kernel


_gather = _make_gather()


@jax.jit
def kernel(token_ids, weight):
  idx = token_ids.reshape(-1).astype(jnp.int32)
  w128 = weight.reshape(VOCAB // 2, 2 * EMBED_DIM)
  out = _gather(idx, w128)
  return out.reshape(BATCH, HIST, EMBED_DIM)


# speedup vs baseline: 1.0626x; 1.0626x over previous
import functools

import jax
import jax.numpy as jnp
from jax import lax
from jax.experimental import pallas as pl
from jax.experimental.pallas import tpu as pltpu
from jax.experimental.pallas import tpu_sc as plsc

VOCAB = 1000000
EMBED_DIM = 64
BATCH = 4096
HIST = 200

NUM_CORES = 2
NUM_SUBCORES = 16
NUM_WORKERS = NUM_CORES * NUM_SUBCORES

B_TOTAL = BATCH * HIST
B_PER_W = B_TOTAL // NUM_WORKERS
ROWS_PER_W = BATCH // NUM_WORKERS
CHUNK = HIST
N_CHUNKS = ROWS_PER_W
NBUF = 4


def _make_gather():
  mesh = plsc.VectorSubcoreMesh(
      core_axis_name="c", subcore_axis_name="s",
      num_cores=NUM_CORES, num_subcores=NUM_SUBCORES)

  @functools.partial(
      pl.kernel,
      mesh=mesh,
      out_type=jax.ShapeDtypeStruct((BATCH, HIST, EMBED_DIM), jnp.float32),
      scratch_types=[
          pltpu.VMEM((B_PER_W,), jnp.int32),
          pltpu.VMEM((NBUF, CHUNK, EMBED_DIM), jnp.float32),
          [pltpu.SemaphoreType.DMA] * NBUF,
          [pltpu.SemaphoreType.DMA] * NBUF,
      ],
      compiler_params=pltpu.CompilerParams(use_tc_tiling_on_sc=False),
  )
  def gather_kernel(idx_hbm, table_hbm, out_hbm, idx_v, rows_v, gsems, ssems):
    wid = lax.axis_index("s") * NUM_CORES + lax.axis_index("c")
    base = wid * B_PER_W
    row_base = wid * ROWS_PER_W
    pltpu.sync_copy(idx_hbm.at[pl.ds(base, B_PER_W)], idx_v)

    @pl.loop(0, N_CHUNKS, step=NBUF)
    def _group(g0):
      for b in range(NBUF):
        g = g0 + b

        @pl.when(g0 > 0)
        def _():
          pltpu.make_async_copy(
              rows_v.at[b], out_hbm.at[0], ssems[b]).wait()

        pltpu.async_copy(
            table_hbm.at[idx_v.at[pl.ds(g * CHUNK, CHUNK)]],
            rows_v.at[b], gsems[b])
      for b in range(NBUF):
        g = g0 + b
        pltpu.make_async_copy(
            table_hbm.at[pl.ds(0, CHUNK)], rows_v.at[b], gsems[b]).wait()
        pltpu.async_copy(
            rows_v.at[b], out_hbm.at[row_base + g], ssems[b])

    for b in range(NBUF):
      pltpu.make_async_copy(
          rows_v.at[b], out_hbm.at[0], ssems[b]).wait()

  return gather_kernel


_gather = _make_gather()


@jax.jit
def kernel(token_ids, weight):
  idx = token_ids.reshape(-1).astype(jnp.int32)
  return _gather(idx, weight)
